# baseline (speedup 1.0000x reference)
import functools

import jax
import jax.numpy as jnp
from jax import lax
from jax.experimental import pallas as pl
from jax.experimental.pallas import tpu as pltpu
from jax.experimental.pallas import tpu_sc as plsc

EMB_DIM = 64
PAIR = 2 * EMB_DIM
BATCH = 16384
L = 16
NC = 2
NS = 16
NW = NC * NS
B_PER_W = BATCH // NW
CHUNK = 128
N_CHUNKS = B_PER_W // CHUNK
BLOCKS = CHUNK // L
PACK_C = 512


def _pack_pairs(table):
    n = table.shape[0]
    k = jnp.arange(PACK_C, dtype=jnp.int32)
    q = jnp.arange(PACK_C // 2, dtype=jnp.int32)
    sel_e = (k[None, :] == 2 * q[:, None]).astype(jnp.float32)
    sel_o = (k[None, :] == 2 * q[:, None] + 1).astype(jnp.float32)

    def body(se_ref, so_ref, x_ref, o_ref):
        x = x_ref[...]
        dn = (((1,), (1,)), ((), ()))
        lo = jax.lax.dot_general(se_ref[...], x, dn,
                                 preferred_element_type=jnp.float32)
        hi = jax.lax.dot_general(so_ref[...], x, dn,
                                 preferred_element_type=jnp.float32)
        o_ref[...] = jnp.concatenate([lo, hi], axis=-1)

    return pl.pallas_call(
        body,
        grid=(pl.cdiv(n, PACK_C),),
        in_specs=[
            pl.BlockSpec((PACK_C // 2, PACK_C), lambda j: (0, 0)),
            pl.BlockSpec((PACK_C // 2, PACK_C), lambda j: (0, 0)),
            pl.BlockSpec((EMB_DIM, PACK_C), lambda j: (0, j)),
        ],
        out_specs=pl.BlockSpec((PACK_C // 2, PAIR), lambda j: (j, 0)),
        out_shape=jax.ShapeDtypeStruct((n // 2, PAIR), jnp.float32),
        compiler_params=pltpu.CompilerParams(
            dimension_semantics=("arbitrary",)),
    )(sel_e, sel_o, table.T)


@functools.partial(
    pl.kernel,
    out_type=jax.ShapeDtypeStruct((BATCH,), jnp.float32),
    mesh=plsc.VectorSubcoreMesh(core_axis_name="c", subcore_axis_name="s"),
    compiler_params=pltpu.CompilerParams(
        needs_layout_passes=False, use_tc_tiling_on_sc=True),
    scratch_types=[
        pltpu.VMEM((N_CHUNKS, CHUNK), jnp.int32),
        pltpu.VMEM((N_CHUNKS, CHUNK), jnp.int32),
        pltpu.VMEM((CHUNK, PAIR), jnp.float32),
        pltpu.VMEM((CHUNK, PAIR), jnp.float32),
        pltpu.VMEM((B_PER_W,), jnp.float32),
        pltpu.VMEM((CHUNK,), jnp.int32),
        pltpu.VMEM((CHUNK,), jnp.int32),
        pltpu.SemaphoreType.DMA,
    ],
)
def _two_tower_sc(u_hbm, p_hbm, ue2_hbm, pe2_hbm, out_hbm,
                  uidx_v, pidx_v, urows, prows, out_v, upair, ppair, sem):
    wid = lax.axis_index("s") * NC + lax.axis_index("c")
    base = wid * B_PER_W

    pltpu.sync_copy(u_hbm.at[wid], uidx_v)
    pltpu.sync_copy(p_hbm.at[wid], pidx_v)

    lane_ids = lax.iota(jnp.int32, L)

    def chunk_body(j, carry):
        for g in range(BLOCKS):
            s = pl.ds(g * L, L)
            upair[s] = jax.lax.shift_right_logical(uidx_v[j, s], 1)
            ppair[s] = jax.lax.shift_right_logical(pidx_v[j, s], 1)
        cu = pltpu.async_copy(ue2_hbm.at[upair], urows, sem)
        cp = pltpu.async_copy(pe2_hbm.at[ppair], prows, sem)
        cu.wait()
        cp.wait()

        for g in range(BLOCKS):
            s = pl.ds(g * L, L)
            rows = g * L + lane_ids
            ubit = (uidx_v[j, s] & 1) * EMB_DIM
            pbit = (pidx_v[j, s] & 1) * EMB_DIM
            acc0 = jnp.zeros((L,), jnp.float32)
            acc1 = jnp.zeros((L,), jnp.float32)
            acc2 = jnp.zeros((L,), jnp.float32)
            acc3 = jnp.zeros((L,), jnp.float32)
            accs = [acc0, acc1, acc2, acc3]
            for d in range(EMB_DIM):
                uv = plsc.load_gather(urows, [rows, ubit + d])
                pv = plsc.load_gather(prows, [rows, pbit + d])
                accs[d % 4] = accs[d % 4] + uv * pv
            dot = (accs[0] + accs[1]) + (accs[2] + accs[3])
            out_v[pl.ds(j * CHUNK + g * L, L)] = 1.0 / (1.0 + jnp.exp(-dot))
        return carry

    lax.fori_loop(0, N_CHUNKS, chunk_body, 0)

    pltpu.sync_copy(out_v, out_hbm.at[pl.ds(base, B_PER_W)])


def kernel(u, p, user_emb, prod_emb):
    u3 = u.astype(jnp.int32).reshape(NW, N_CHUNKS, CHUNK)
    p3 = p.astype(jnp.int32).reshape(NW, N_CHUNKS, CHUNK)
    ue2 = _pack_pairs(user_emb)
    pe2 = _pack_pairs(prod_emb)
    return _two_tower_sc(u3, p3, ue2, pe2)


# speedup vs baseline: 1.6482x; 1.6482x over previous
import functools

import jax
import jax.numpy as jnp
from jax import lax
from jax.experimental import pallas as pl
from jax.experimental.pallas import tpu as pltpu
from jax.experimental.pallas import tpu_sc as plsc

EMB_DIM = 64
PAIR = 2 * EMB_DIM
BATCH = 16384
L = 16
NC = 2
NS = 16
NW = NC * NS
B_PER_W = BATCH // NW
CHUNK = 128
N_CHUNKS = B_PER_W // CHUNK
BLOCKS = CHUNK // L
PACK_C = 2048


def _pack_pairs(table):
    n = table.shape[0]
    eye = jnp.eye(EMB_DIM, dtype=jnp.float32)

    def body(i_ref, x_ref, o_ref):
        x = x_ref[...]
        t = jax.lax.dot_general(
            x, i_ref[...], (((0,), (0,)), ((), ())),
            preferred_element_type=jnp.float32,
            precision=jax.lax.Precision.HIGHEST)
        t3 = t.reshape(PACK_C // 2, 2, EMB_DIM)
        o_ref[...] = jnp.concatenate([t3[:, 0, :], t3[:, 1, :]], axis=-1)

    return pl.pallas_call(
        body,
        grid=(pl.cdiv(n, PACK_C),),
        in_specs=[
            pl.BlockSpec((EMB_DIM, EMB_DIM), lambda j: (0, 0)),
            pl.BlockSpec((EMB_DIM, PACK_C), lambda j: (0, j)),
        ],
        out_specs=pl.BlockSpec((PACK_C // 2, PAIR), lambda j: (j, 0)),
        out_shape=jax.ShapeDtypeStruct((n // 2, PAIR), jnp.float32),
        compiler_params=pltpu.CompilerParams(
            dimension_semantics=("arbitrary",)),
    )(eye, table.T)


@functools.partial(
    pl.kernel,
    out_type=jax.ShapeDtypeStruct((BATCH,), jnp.float32),
    mesh=plsc.VectorSubcoreMesh(core_axis_name="c", subcore_axis_name="s"),
    compiler_params=pltpu.CompilerParams(
        needs_layout_passes=False, use_tc_tiling_on_sc=True),
    scratch_types=[
        pltpu.VMEM((N_CHUNKS, CHUNK), jnp.int32),
        pltpu.VMEM((N_CHUNKS, CHUNK), jnp.int32),
        pltpu.VMEM((CHUNK, PAIR), jnp.float32),
        pltpu.VMEM((CHUNK, PAIR), jnp.float32),
        pltpu.VMEM((B_PER_W,), jnp.float32),
        pltpu.VMEM((CHUNK,), jnp.int32),
        pltpu.VMEM((CHUNK,), jnp.int32),
        pltpu.SemaphoreType.DMA,
    ],
)
def _two_tower_sc(u_hbm, p_hbm, ue2_hbm, pe2_hbm, out_hbm,
                  uidx_v, pidx_v, urows, prows, out_v, upair, ppair, sem):
    wid = lax.axis_index("s") * NC + lax.axis_index("c")
    base = wid * B_PER_W

    pltpu.sync_copy(u_hbm.at[wid], uidx_v)
    pltpu.sync_copy(p_hbm.at[wid], pidx_v)

    lane_ids = lax.iota(jnp.int32, L)

    def chunk_body(j, carry):
        for g in range(BLOCKS):
            s = pl.ds(g * L, L)
            upair[s] = jax.lax.shift_right_logical(uidx_v[j, s], 1)
            ppair[s] = jax.lax.shift_right_logical(pidx_v[j, s], 1)
        cu = pltpu.async_copy(ue2_hbm.at[upair], urows, sem)
        cp = pltpu.async_copy(pe2_hbm.at[ppair], prows, sem)
        cu.wait()
        cp.wait()

        for g in range(BLOCKS):
            s = pl.ds(g * L, L)
            rows = g * L + lane_ids
            ubit = (uidx_v[j, s] & 1) * EMB_DIM
            pbit = (pidx_v[j, s] & 1) * EMB_DIM
            acc0 = jnp.zeros((L,), jnp.float32)
            acc1 = jnp.zeros((L,), jnp.float32)
            acc2 = jnp.zeros((L,), jnp.float32)
            acc3 = jnp.zeros((L,), jnp.float32)
            accs = [acc0, acc1, acc2, acc3]
            for d in range(EMB_DIM):
                uv = plsc.load_gather(urows, [rows, ubit + d])
                pv = plsc.load_gather(prows, [rows, pbit + d])
                accs[d % 4] = accs[d % 4] + uv * pv
            dot = (accs[0] + accs[1]) + (accs[2] + accs[3])
            out_v[pl.ds(j * CHUNK + g * L, L)] = 1.0 / (1.0 + jnp.exp(-dot))
        return carry

    lax.fori_loop(0, N_CHUNKS, chunk_body, 0)

    pltpu.sync_copy(out_v, out_hbm.at[pl.ds(base, B_PER_W)])


def kernel(u, p, user_emb, prod_emb):
    u3 = u.astype(jnp.int32).reshape(NW, N_CHUNKS, CHUNK)
    p3 = p.astype(jnp.int32).reshape(NW, N_CHUNKS, CHUNK)
    ue2 = _pack_pairs(user_emb)
    pe2 = _pack_pairs(prod_emb)
    return _two_tower_sc(u3, p3, ue2, pe2)


# speedup vs baseline: 2.3951x; 1.4532x over previous
import functools

import jax
import jax.numpy as jnp
from jax import lax
from jax.experimental import pallas as pl
from jax.experimental.pallas import tpu as pltpu
from jax.experimental.pallas import tpu_sc as plsc

EMB_DIM = 64
ROW_PAD = 128
BATCH = 16384
L = 16
NC = 2
NS = 16
NW = NC * NS
B_PER_W = BATCH // NW
CHUNK = 128
N_CHUNKS = B_PER_W // CHUNK
BLOCKS = CHUNK // L


@functools.partial(
    pl.kernel,
    out_type=jax.ShapeDtypeStruct((BATCH,), jnp.float32),
    mesh=plsc.VectorSubcoreMesh(core_axis_name="c", subcore_axis_name="s"),
    compiler_params=pltpu.CompilerParams(
        needs_layout_passes=False, use_tc_tiling_on_sc=True),
    scratch_types=[
        pltpu.VMEM((N_CHUNKS, CHUNK), jnp.int32),
        pltpu.VMEM((N_CHUNKS, CHUNK), jnp.int32),
        pltpu.VMEM((CHUNK, ROW_PAD), jnp.float32),
        pltpu.VMEM((CHUNK, ROW_PAD), jnp.float32),
        pltpu.VMEM((B_PER_W,), jnp.float32),
        pltpu.SemaphoreType.DMA,
    ],
)
def _two_tower_sc(u_hbm, p_hbm, uep_hbm, pep_hbm, out_hbm,
                  uidx_v, pidx_v, urows, prows, out_v, sem):
    wid = lax.axis_index("s") * NC + lax.axis_index("c")
    base = wid * B_PER_W

    pltpu.sync_copy(u_hbm.at[wid], uidx_v)
    pltpu.sync_copy(p_hbm.at[wid], pidx_v)

    lane_ids = lax.iota(jnp.int32, L)

    def chunk_body(j, carry):
        cu = pltpu.async_copy(uep_hbm.at[uidx_v.at[j]], urows, sem)
        cp = pltpu.async_copy(pep_hbm.at[pidx_v.at[j]], prows, sem)
        cu.wait()
        cp.wait()

        for g in range(BLOCKS):
            rows = g * L + lane_ids
            acc0 = jnp.zeros((L,), jnp.float32)
            acc1 = jnp.zeros((L,), jnp.float32)
            acc2 = jnp.zeros((L,), jnp.float32)
            acc3 = jnp.zeros((L,), jnp.float32)
            accs = [acc0, acc1, acc2, acc3]
            for d in range(EMB_DIM):
                dvec = jnp.full((L,), d, jnp.int32)
                uv = plsc.load_gather(urows, [rows, dvec])
                pv = plsc.load_gather(prows, [rows, dvec])
                accs[d % 4] = accs[d % 4] + uv * pv
            dot = (accs[0] + accs[1]) + (accs[2] + accs[3])
            out_v[pl.ds(j * CHUNK + g * L, L)] = 1.0 / (1.0 + jnp.exp(-dot))
        return carry

    lax.fori_loop(0, N_CHUNKS, chunk_body, 0)

    pltpu.sync_copy(out_v, out_hbm.at[pl.ds(base, B_PER_W)])


def kernel(u, p, user_emb, prod_emb):
    u3 = u.astype(jnp.int32).reshape(NW, N_CHUNKS, CHUNK)
    p3 = p.astype(jnp.int32).reshape(NW, N_CHUNKS, CHUNK)
    uep = jnp.pad(user_emb, ((0, 0), (0, ROW_PAD - EMB_DIM)))
    pep = jnp.pad(prod_emb, ((0, 0), (0, ROW_PAD - EMB_DIM)))
    return _two_tower_sc(u3, p3, uep, pep)
